# baseline (speedup 1.0000x reference)
import functools
import jax
import jax.numpy as jnp
import numpy as np
from jax import lax
from jax.experimental import pallas as pl
from jax.experimental.pallas import tpu as pltpu
from jax.experimental.pallas import tpu_sc as plsc

S = 2048
D = 1024
H = 16
KV = 2
DH = 64
E = 8
DFF = 2048
EPS = 1e-05

SB = 256
BLK = 256
NB = 24
NPAD = NB * BLK
NA = 2 * S
NW = 32
RPW = NPAD // NW


def _rms(v, scale, eps=EPS):
    ms = jnp.mean(v * v, axis=-1, keepdims=True)
    return v * lax.rsqrt(ms + eps) * scale


def _f32(v):
    return v.astype(jnp.float32)


def _bf(v):
    return _f32(v.astype(jnp.bfloat16))


def _segsum(p, Sseg):
    hi = _bf(p)
    r1 = p - hi
    hi2 = _bf(r1)
    hi3 = _bf(r1 - hi2)
    acc = jnp.dot(hi, Sseg, preferred_element_type=jnp.float32)
    acc += jnp.dot(hi2, Sseg, preferred_element_type=jnp.float32)
    acc += jnp.dot(hi3, Sseg, preferred_element_type=jnp.float32)
    return acc


def _attn_body(xb_ref, pres_ref, posts_ref, premoe_ref, Wq_ref, Wk_ref,
               Wv_ref, Wo_ref, Wr_ref, y_ref, xn2b_ref, w_ref, mask_ref):
    xb = xb_ref[...]
    xn = _rms(xb, pres_ref[...])
    q = jnp.dot(xn, Wq_ref[...], preferred_element_type=jnp.float32)
    kk = jnp.dot(xn, Wk_ref[...], preferred_element_type=jnp.float32)
    vv = jnp.dot(xn, Wv_ref[...], preferred_element_type=jnp.float32)
    k0, k1 = kk[:, :DH], kk[:, DH:]
    v0, v1 = vv[:, :DH], vv[:, DH:]
    krep0 = jnp.concatenate([k0] * H, axis=1)
    krep1 = jnp.concatenate([k1] * H, axis=1)
    r_i = lax.broadcasted_iota(jnp.int32, (H * DH, H), 0)
    c_i = lax.broadcasted_iota(jnp.int32, (H * DH, H), 1)
    Sseg = ((r_i // DH) == c_i).astype(jnp.float32)
    scale = 1.0 / np.sqrt(DH)
    qb = _bf(q)
    A0 = _segsum(qb * _bf(krep0), Sseg) * scale
    A1 = _segsum(qb * _bf(krep1), Sseg) * scale
    m = jnp.maximum(A0, A1)
    h_i = lax.broadcasted_iota(jnp.int32, (SB, H), 1)
    n0 = jnp.minimum(h_i + 1, 8).astype(jnp.float32)
    n1 = jnp.maximum(h_i - 7, 0).astype(jnp.float32)
    e0 = jnp.exp(A0 - m)
    e1 = jnp.exp(A1 - m)
    z = n0 * e0 + n1 * e1
    w0 = e0 / z
    w1 = e1 / z
    rr = lax.broadcasted_iota(jnp.int32, (H, H * DH), 0)
    cc = lax.broadcasted_iota(jnp.int32, (H, H * DH), 1)
    Eseg = ((cc // DH) == rr).astype(jnp.float32)
    W0 = jnp.dot(_bf(w0), Eseg, preferred_element_type=jnp.float32)
    W1 = jnp.dot(_bf(w1), Eseg, preferred_element_type=jnp.float32)
    vb0 = _bf(jnp.concatenate([v0] * H, axis=1))
    vb1 = _bf(jnp.concatenate([v1] * H, axis=1))
    l_i = lax.broadcasted_iota(jnp.int32, (SB, H * DH), 1) // DH
    n0f = jnp.minimum(l_i + 1, 8).astype(jnp.float32)
    n1f = jnp.maximum(l_i - 7, 0).astype(jnp.float32)
    attn = n0f * (W0 * vb0) + n1f * (W1 * vb1)
    mqa = jnp.dot(attn, Wo_ref[...], preferred_element_type=jnp.float32)
    y = xb + _rms(mqa, posts_ref[...])
    y_ref[...] = y
    xn2 = _rms(y, premoe_ref[...])
    xn2b_ref[...] = xn2.astype(jnp.bfloat16)
    logits = jnp.dot(xn2, Wr_ref[...], preferred_element_type=jnp.float32)
    lm = jnp.max(logits, axis=1, keepdims=True)
    p = jnp.exp(logits - lm)
    p = p / jnp.sum(p, axis=1, keepdims=True)
    e_i = lax.broadcasted_iota(jnp.int32, (SB, E), 1)
    m1 = jnp.max(p, axis=1, keepdims=True)
    i1 = jnp.min(jnp.where(p == m1, e_i, E), axis=1, keepdims=True)
    p2 = jnp.where(e_i == i1, -1.0, p)
    m2 = jnp.max(p2, axis=1, keepdims=True)
    i2 = jnp.min(jnp.where(p2 == m2, e_i, E), axis=1, keepdims=True)
    tv1 = m1 / (m1 + m2)
    tv2 = m2 / (m1 + m2)
    w_all = (jnp.where(e_i == i1, tv1, 0.0) + jnp.where(e_i == i2, tv2, 0.0))
    w_ref[...] = w_all
    mask_ref[...] = (w_all > 0).astype(jnp.float32)


def _route_body(w_ref, pos_ref, gates_ref, be_ref):
    w_all = w_ref[...]
    maskb = w_all > 0.0
    maskf = maskb.astype(jnp.float32)
    r_i = lax.broadcasted_iota(jnp.int32, (S, S), 0)
    c_i = lax.broadcasted_iota(jnp.int32, (S, S), 1)
    L = (c_i < r_i).astype(jnp.float32)
    rank = jnp.dot(L, maskf, preferred_element_type=jnp.float32)
    counts = jnp.sum(maskf, axis=0, keepdims=True)
    nb = jnp.floor((counts + (BLK - 1)) * (1.0 / BLK))
    rr = lax.broadcasted_iota(jnp.int32, (E, E), 0)
    cc = lax.broadcasted_iota(jnp.int32, (E, E), 1)
    T8 = (rr < cc).astype(jnp.float32)
    starts = jnp.dot(nb, T8, preferred_element_type=jnp.float32)
    off = starts * float(BLK)
    posf = off + rank
    e_i = lax.broadcasted_iota(jnp.int32, (S, E), 1)
    e1 = jnp.min(jnp.where(maskb, e_i, E), axis=1, keepdims=True)
    e2 = jnp.max(jnp.where(maskb, e_i, -1), axis=1, keepdims=True)
    sel1 = e_i == e1
    sel2 = e_i == e2
    p1 = jnp.sum(jnp.where(sel1, posf, 0.0), axis=1, keepdims=True)
    p2 = jnp.sum(jnp.where(sel2, posf, 0.0), axis=1, keepdims=True)
    g1 = jnp.sum(jnp.where(sel1, w_all, 0.0), axis=1, keepdims=True)
    g2 = jnp.sum(jnp.where(sel2, w_all, 0.0), axis=1, keepdims=True)
    pos_ref[...] = jnp.concatenate([p1, p2], axis=1).astype(jnp.int32)
    gates_ref[...] = jnp.concatenate([g1, g2], axis=1)
    b_i = lax.broadcasted_iota(jnp.int32, (NB, E), 0).astype(jnp.float32)
    ge = (b_i >= starts).astype(jnp.float32)
    be = jnp.sum(ge, axis=1, keepdims=True) - 1.0
    be_ref[...] = jnp.broadcast_to(be, (NB, E))


_SC_MESH = plsc.VectorSubcoreMesh(core_axis_name="c", subcore_axis_name="s")
_APT = NA // 16
_ZPT = NPAD // 16


@functools.partial(
    pl.kernel, mesh=_SC_MESH,
    out_type=jax.ShapeDtypeStruct((NPAD,), jnp.int32),
    scratch_types=[
        pltpu.VMEM((4, 64), jnp.int32),
        pltpu.VMEM((4, 64), jnp.int32),
        pltpu.VMEM((_ZPT,), jnp.int32),
        pltpu.SemaphoreType.DMA,
    ],
)
def _sc_scatter_tid(pos_hbm, tid_hbm, idx_v, val_v, z_v, sem):
    c = lax.axis_index("c")
    s = lax.axis_index("s")

    @pl.when(c == 0)
    def _():
        for j in range(_ZPT // 16):
            z_v[pl.ds(j * 16, 16)] = jnp.zeros((16,), jnp.int32)
        pltpu.sync_copy(z_v, tid_hbm.at[pl.ds(s * _ZPT, _ZPT)])
        base = s * _APT
        pltpu.sync_copy(pos_hbm.at[pl.ds(s * 4, 4)], idx_v)
        for j in range(_APT // 16):
            a = base + j * 16 + lax.iota(jnp.int32, 16)
            val_v[j // 4, pl.ds((j % 4) * 16, 16)] = a >> 1
        plsc.subcore_barrier()
        for j in range(4):
            pltpu.async_copy(val_v.at[j], tid_hbm.at[idx_v.at[j]], sem).wait()


_GC = RPW // 2


@functools.partial(
    pl.kernel, mesh=_SC_MESH,
    out_type=jax.ShapeDtypeStruct((NPAD, D // 256, 128), jnp.int32),
    scratch_types=[
        pltpu.VMEM((2, _GC), jnp.int32),
        pltpu.VMEM((_GC, D // 256, 128), jnp.int32),
        pltpu.VMEM((_GC, D // 256, 128), jnp.int32),
        pltpu.SemaphoreType.DMA,
        pltpu.SemaphoreType.DMA,
        pltpu.SemaphoreType.DMA,
        pltpu.SemaphoreType.DMA,
    ],
)
def _sc_gather_rows(tid2d_hbm, xnb_hbm, xg_hbm, idx_v, buf0, buf1,
                    gs0, gs1, ws0, ws1):
    c = lax.axis_index("c")
    s = lax.axis_index("s")
    wid = s * 2 + c
    base = wid * RPW
    pltpu.sync_copy(tid2d_hbm.at[pl.ds(wid * 2, 2)], idx_v)
    g0 = pltpu.async_copy(xnb_hbm.at[idx_v.at[0]], buf0, gs0)
    g1 = pltpu.async_copy(xnb_hbm.at[idx_v.at[1]], buf1, gs1)
    g0.wait()
    w0 = pltpu.async_copy(buf0, xg_hbm.at[pl.ds(base, _GC)], ws0)
    g1.wait()
    w1 = pltpu.async_copy(buf1, xg_hbm.at[pl.ds(base + _GC, _GC)], ws1)
    w0.wait()
    w1.wait()


@functools.partial(
    pl.kernel, mesh=_SC_MESH,
    out_type=[
        jax.ShapeDtypeStruct((S, D // 256, 128), jnp.int32),
        jax.ShapeDtypeStruct((S, D // 256, 128), jnp.int32),
    ],
    scratch_types=[
        pltpu.VMEM((1, 64), jnp.int32),
        pltpu.VMEM((1, 64), jnp.int32),
        pltpu.VMEM((64, D // 256, 128), jnp.int32),
        pltpu.VMEM((64, D // 256, 128), jnp.int32),
        pltpu.SemaphoreType.DMA,
        pltpu.SemaphoreType.DMA,
        pltpu.SemaphoreType.DMA,
        pltpu.SemaphoreType.DMA,
    ],
)
def _sc_gather_back(p1_hbm, p2_hbm, h_hbm, g1_hbm, g2_hbm, idx1, idx2,
                    buf0, buf1, gs0, gs1, ws0, ws1):
    c = lax.axis_index("c")
    s = lax.axis_index("s")
    wid = s * 2 + c
    pltpu.sync_copy(p1_hbm.at[pl.ds(wid, 1)], idx1)
    pltpu.sync_copy(p2_hbm.at[pl.ds(wid, 1)], idx2)
    g0 = pltpu.async_copy(h_hbm.at[idx1.at[0]], buf0, gs0)
    g1 = pltpu.async_copy(h_hbm.at[idx2.at[0]], buf1, gs1)
    g0.wait()
    w0 = pltpu.async_copy(buf0, g1_hbm.at[pl.ds(wid * 64, 64)], ws0)
    g1.wait()
    w1 = pltpu.async_copy(buf1, g2_hbm.at[pl.ds(wid * 64, 64)], ws1)
    w0.wait()
    w1.wait()


def _expert_body(be_ref, xg_ref, w1_ref, w2_ref, h_ref):
    h = jnp.dot(xg_ref[...], w1_ref[0], preferred_element_type=jnp.float32)
    h = h / (1.0 + jnp.exp(-h))
    h_ref[...] = jnp.dot(h.astype(jnp.bfloat16), w2_ref[0],
                         preferred_element_type=jnp.float32
                         ).astype(jnp.bfloat16)


def _final_body(y_ref, g1_ref, g2_ref, gates_ref, postmoe_ref, out_ref):
    g = gates_ref[...]
    moe = g[:, 0:1] * _f32(g1_ref[...]) + g[:, 1:2] * _f32(g2_ref[...])
    out_ref[...] = y_ref[...] + _rms(moe, postmoe_ref[...])


def kernel(x, pre_mqa_scale, post_mqa_scale, pre_moe_scale, post_moe_scale,
           Wq, Wk, Wv, Wo, Wr, w1, w2):
    xs = x.reshape(S, D).astype(jnp.float32)
    pres = pre_mqa_scale.reshape(1, D)
    posts = post_mqa_scale.reshape(1, D)
    premoe = pre_moe_scale.reshape(1, D)
    postmoe = post_moe_scale.reshape(1, D)
    w1b = w1.astype(jnp.bfloat16)
    w2b = w2.astype(jnp.bfloat16)

    full = lambda shape: pl.BlockSpec(shape, lambda i: (0,) * len(shape))
    y, xnb, w_all, mask = pl.pallas_call(
        _attn_body,
        grid=(S // SB,),
        in_specs=[
            pl.BlockSpec((SB, D), lambda i: (i, 0)),
            full((1, D)), full((1, D)), full((1, D)),
            full((D, H * DH)), full((D, KV * DH)), full((D, KV * DH)),
            full((H * DH, D)), full((D, E)),
        ],
        out_specs=[
            pl.BlockSpec((SB, D), lambda i: (i, 0)),
            pl.BlockSpec((SB, D), lambda i: (i, 0)),
            pl.BlockSpec((SB, E), lambda i: (i, 0)),
            pl.BlockSpec((SB, E), lambda i: (i, 0)),
        ],
        out_shape=[
            jax.ShapeDtypeStruct((S, D), jnp.float32),
            jax.ShapeDtypeStruct((S, D), jnp.bfloat16),
            jax.ShapeDtypeStruct((S, E), jnp.float32),
            jax.ShapeDtypeStruct((S, E), jnp.float32),
        ],
    )(xs, pres, posts, premoe, Wq, Wk, Wv, Wo, Wr)

    pos2, gates2, beM = pl.pallas_call(
        _route_body,
        grid=(1,),
        in_specs=[full((S, E))],
        out_specs=[full((S, 2)), full((S, 2)), full((NB, E))],
        out_shape=[
            jax.ShapeDtypeStruct((S, 2), jnp.int32),
            jax.ShapeDtypeStruct((S, 2), jnp.float32),
            jax.ShapeDtypeStruct((NB, E), jnp.float32),
        ],
    )(w_all)

    blk_expert = beM[:, 0].astype(jnp.int32)
    pos2d = pos2.reshape(NA // 64, 64)

    sorted_tid = _sc_scatter_tid(pos2d)
    xnb_i = lax.bitcast_convert_type(
        xnb.reshape(S, D // 2, 2), jnp.int32).reshape(S, D // 256, 128)
    xg3 = _sc_gather_rows(sorted_tid.reshape(NPAD // _GC, _GC), xnb_i)
    xg = lax.bitcast_convert_type(
        xg3.reshape(NPAD, D // 2), jnp.bfloat16).reshape(NPAD, D)

    grid_spec = pltpu.PrefetchScalarGridSpec(
        num_scalar_prefetch=1,
        grid=(NB,),
        in_specs=[
            pl.BlockSpec((BLK, D), lambda i, be: (i, 0)),
            pl.BlockSpec((1, D, DFF), lambda i, be: (be[i], 0, 0)),
            pl.BlockSpec((1, DFF, D), lambda i, be: (be[i], 0, 0)),
        ],
        out_specs=pl.BlockSpec((BLK, D), lambda i, be: (i, 0)),
    )
    hrows = pl.pallas_call(
        _expert_body,
        grid_spec=grid_spec,
        out_shape=jax.ShapeDtypeStruct((NPAD, D), jnp.bfloat16),
    )(blk_expert, xg, w1b, w2b)

    hrows_i = lax.bitcast_convert_type(
        hrows.reshape(NPAD, D // 2, 2), jnp.int32).reshape(NPAD, D // 256, 128)
    g1, g2 = _sc_gather_back(
        pos2[:, 0].reshape(S // 64, 64), pos2[:, 1].reshape(S // 64, 64),
        hrows_i)
    g1 = lax.bitcast_convert_type(
        g1.reshape(S, D // 2), jnp.bfloat16).reshape(S, D)
    g2 = lax.bitcast_convert_type(
        g2.reshape(S, D // 2), jnp.bfloat16).reshape(S, D)

    out = pl.pallas_call(
        _final_body,
        grid=(S // SB,),
        in_specs=[
            pl.BlockSpec((SB, D), lambda i: (i, 0)),
            pl.BlockSpec((SB, D), lambda i: (i, 0)),
            pl.BlockSpec((SB, D), lambda i: (i, 0)),
            pl.BlockSpec((SB, 2), lambda i: (i, 0)),
            full((1, D)),
        ],
        out_specs=pl.BlockSpec((SB, D), lambda i: (i, 0)),
        out_shape=jax.ShapeDtypeStruct((S, D), jnp.float32),
    )(y, g1, g2, gates2, postmoe)

    return out.reshape(1, S, D), mask


# speedup vs baseline: 3.2560x; 3.2560x over previous
import functools
import jax
import jax.numpy as jnp
import numpy as np
from jax import lax
from jax.experimental import pallas as pl
from jax.experimental.pallas import tpu as pltpu
from jax.experimental.pallas import tpu_sc as plsc

S = 2048
D = 1024
H = 16
KV = 2
DH = 64
E = 8
DFF = 2048
EPS = 1e-05

SB = 256
BLK = 256
NB = 24
NPAD = NB * BLK
NA = 2 * S
NW = 32
RPW = NPAD // NW


def _rms(v, scale, eps=EPS):
    ms = jnp.mean(v * v, axis=-1, keepdims=True)
    return v * lax.rsqrt(ms + eps) * scale


def _f32(v):
    return v.astype(jnp.float32)


def _bf(v):
    return _f32(v.astype(jnp.bfloat16))


def _segsum(p, Sseg):
    hi = _bf(p)
    r1 = p - hi
    hi2 = _bf(r1)
    hi3 = _bf(r1 - hi2)
    acc = jnp.dot(hi, Sseg, preferred_element_type=jnp.float32)
    acc += jnp.dot(hi2, Sseg, preferred_element_type=jnp.float32)
    acc += jnp.dot(hi3, Sseg, preferred_element_type=jnp.float32)
    return acc


def _attn_body(xb_ref, pres_ref, posts_ref, premoe_ref, Wq_ref, Wk_ref,
               Wv_ref, Wo_ref, Wr_ref, y_ref, xn2b_ref, w_ref, mask_ref):
    xb = xb_ref[...]
    xn = _rms(xb, pres_ref[...])
    q = jnp.dot(xn, Wq_ref[...], preferred_element_type=jnp.float32)
    kk = jnp.dot(xn, Wk_ref[...], preferred_element_type=jnp.float32)
    vv = jnp.dot(xn, Wv_ref[...], preferred_element_type=jnp.float32)
    k0, k1 = kk[:, :DH], kk[:, DH:]
    v0, v1 = vv[:, :DH], vv[:, DH:]
    krep0 = jnp.concatenate([k0] * H, axis=1)
    krep1 = jnp.concatenate([k1] * H, axis=1)
    r_i = lax.broadcasted_iota(jnp.int32, (H * DH, H), 0)
    c_i = lax.broadcasted_iota(jnp.int32, (H * DH, H), 1)
    Sseg = ((r_i // DH) == c_i).astype(jnp.float32)
    scale = 1.0 / np.sqrt(DH)
    qb = _bf(q)
    A0 = _segsum(qb * _bf(krep0), Sseg) * scale
    A1 = _segsum(qb * _bf(krep1), Sseg) * scale
    m = jnp.maximum(A0, A1)
    h_i = lax.broadcasted_iota(jnp.int32, (SB, H), 1)
    n0 = jnp.minimum(h_i + 1, 8).astype(jnp.float32)
    n1 = jnp.maximum(h_i - 7, 0).astype(jnp.float32)
    e0 = jnp.exp(A0 - m)
    e1 = jnp.exp(A1 - m)
    z = n0 * e0 + n1 * e1
    w0 = e0 / z
    w1 = e1 / z
    rr = lax.broadcasted_iota(jnp.int32, (H, H * DH), 0)
    cc = lax.broadcasted_iota(jnp.int32, (H, H * DH), 1)
    Eseg = ((cc // DH) == rr).astype(jnp.float32)
    W0 = jnp.dot(_bf(w0), Eseg, preferred_element_type=jnp.float32)
    W1 = jnp.dot(_bf(w1), Eseg, preferred_element_type=jnp.float32)
    vb0 = _bf(jnp.concatenate([v0] * H, axis=1))
    vb1 = _bf(jnp.concatenate([v1] * H, axis=1))
    l_i = lax.broadcasted_iota(jnp.int32, (SB, H * DH), 1) // DH
    n0f = jnp.minimum(l_i + 1, 8).astype(jnp.float32)
    n1f = jnp.maximum(l_i - 7, 0).astype(jnp.float32)
    attn = n0f * (W0 * vb0) + n1f * (W1 * vb1)
    mqa = jnp.dot(attn, Wo_ref[...], preferred_element_type=jnp.float32)
    y = xb + _rms(mqa, posts_ref[...])
    y_ref[...] = y
    xn2 = _rms(y, premoe_ref[...])
    lo = lax.bitcast_convert_type(_bf(xn2[:, :D // 2]), jnp.int32)
    hi = lax.bitcast_convert_type(_bf(xn2[:, D // 2:]), jnp.int32)
    xn2b_ref[...] = lax.shift_right_logical(lo, 16) | hi
    logits = jnp.dot(xn2, Wr_ref[...], preferred_element_type=jnp.float32)
    lm = jnp.max(logits, axis=1, keepdims=True)
    p = jnp.exp(logits - lm)
    p = p / jnp.sum(p, axis=1, keepdims=True)
    e_i = lax.broadcasted_iota(jnp.int32, (SB, E), 1)
    m1 = jnp.max(p, axis=1, keepdims=True)
    i1 = jnp.min(jnp.where(p == m1, e_i, E), axis=1, keepdims=True)
    p2 = jnp.where(e_i == i1, -1.0, p)
    m2 = jnp.max(p2, axis=1, keepdims=True)
    i2 = jnp.min(jnp.where(p2 == m2, e_i, E), axis=1, keepdims=True)
    tv1 = m1 / (m1 + m2)
    tv2 = m2 / (m1 + m2)
    w_all = (jnp.where(e_i == i1, tv1, 0.0) + jnp.where(e_i == i2, tv2, 0.0))
    w_ref[...] = w_all
    mask_ref[...] = (w_all > 0).astype(jnp.float32)


def _route_body(w_ref, pos_ref, gates_ref, be_ref):
    w_all = w_ref[...]
    maskb = w_all > 0.0
    maskf = maskb.astype(jnp.float32)
    r_i = lax.broadcasted_iota(jnp.int32, (S, S), 0)
    c_i = lax.broadcasted_iota(jnp.int32, (S, S), 1)
    L = (c_i < r_i).astype(jnp.float32)
    rank = jnp.dot(L, maskf, preferred_element_type=jnp.float32)
    counts = jnp.sum(maskf, axis=0, keepdims=True)
    nb = jnp.floor((counts + (BLK - 1)) * (1.0 / BLK))
    rr = lax.broadcasted_iota(jnp.int32, (E, E), 0)
    cc = lax.broadcasted_iota(jnp.int32, (E, E), 1)
    T8 = (rr < cc).astype(jnp.float32)
    starts = jnp.dot(nb, T8, preferred_element_type=jnp.float32)
    off = starts * float(BLK)
    posf = off + rank
    e_i = lax.broadcasted_iota(jnp.int32, (S, E), 1)
    e1 = jnp.min(jnp.where(maskb, e_i, E), axis=1, keepdims=True)
    e2 = jnp.max(jnp.where(maskb, e_i, -1), axis=1, keepdims=True)
    sel1 = e_i == e1
    sel2 = e_i == e2
    p1 = jnp.sum(jnp.where(sel1, posf, 0.0), axis=1, keepdims=True)
    p2 = jnp.sum(jnp.where(sel2, posf, 0.0), axis=1, keepdims=True)
    g1 = jnp.sum(jnp.where(sel1, w_all, 0.0), axis=1, keepdims=True)
    g2 = jnp.sum(jnp.where(sel2, w_all, 0.0), axis=1, keepdims=True)
    pos_ref[...] = jnp.concatenate([p1, p2], axis=1).astype(jnp.int32)
    gates_ref[...] = jnp.concatenate([g1, g2], axis=1)
    b_i = lax.broadcasted_iota(jnp.int32, (NB, E), 0).astype(jnp.float32)
    ge = (b_i >= starts).astype(jnp.float32)
    be = jnp.sum(ge, axis=1, keepdims=True) - 1.0
    be_ref[...] = jnp.broadcast_to(be, (NB, E))


_SC_MESH = plsc.VectorSubcoreMesh(core_axis_name="c", subcore_axis_name="s")
DP = D // 2
TPW = S // NW


@functools.partial(
    pl.kernel, mesh=_SC_MESH,
    out_type=jax.ShapeDtypeStruct((NPAD, DP), jnp.int32),
    scratch_types=[
        pltpu.VMEM((1, TPW), jnp.int32),
        pltpu.VMEM((1, TPW), jnp.int32),
        pltpu.VMEM((TPW, DP), jnp.int32),
        pltpu.SemaphoreType.DMA,
        pltpu.SemaphoreType.DMA,
        pltpu.SemaphoreType.DMA,
        pltpu.SemaphoreType.DMA,
        pltpu.SemaphoreType.DMA,
    ],
)
def _sc_scatter_rows(p1_hbm, p2_hbm, xp_hbm, xg_hbm, idx1, idx2, buf,
                     ls, is1, is2, ws1, ws2):
    c = lax.axis_index("c")
    s = lax.axis_index("s")
    wid = s * 2 + c
    l0 = pltpu.async_copy(xp_hbm.at[pl.ds(wid * TPW, TPW)], buf, ls)
    l1 = pltpu.async_copy(p1_hbm.at[pl.ds(wid, 1)], idx1, is1)
    l2 = pltpu.async_copy(p2_hbm.at[pl.ds(wid, 1)], idx2, is2)
    l0.wait()
    l1.wait()
    l2.wait()
    w1 = pltpu.async_copy(buf, xg_hbm.at[idx1.at[0]], ws1)
    w2 = pltpu.async_copy(buf, xg_hbm.at[idx2.at[0]], ws2)
    w1.wait()
    w2.wait()


@functools.partial(
    pl.kernel, mesh=_SC_MESH,
    out_type=[
        jax.ShapeDtypeStruct((S, DP), jnp.int32),
        jax.ShapeDtypeStruct((S, DP), jnp.int32),
    ],
    scratch_types=[
        pltpu.VMEM((1, TPW), jnp.int32),
        pltpu.VMEM((1, TPW), jnp.int32),
        pltpu.VMEM((TPW, DP), jnp.int32),
        pltpu.VMEM((TPW, DP), jnp.int32),
        pltpu.SemaphoreType.DMA,
        pltpu.SemaphoreType.DMA,
        pltpu.SemaphoreType.DMA,
        pltpu.SemaphoreType.DMA,
        pltpu.SemaphoreType.DMA,
        pltpu.SemaphoreType.DMA,
    ],
)
def _sc_gather_back(p1_hbm, p2_hbm, h_hbm, g1_hbm, g2_hbm, idx1, idx2,
                    buf0, buf1, is1, is2, gs0, gs1, ws0, ws1):
    c = lax.axis_index("c")
    s = lax.axis_index("s")
    wid = s * 2 + c
    l1 = pltpu.async_copy(p1_hbm.at[pl.ds(wid, 1)], idx1, is1)
    l2 = pltpu.async_copy(p2_hbm.at[pl.ds(wid, 1)], idx2, is2)
    l1.wait()
    l2.wait()
    g0 = pltpu.async_copy(h_hbm.at[idx1.at[0]], buf0, gs0)
    g1c = pltpu.async_copy(h_hbm.at[idx2.at[0]], buf1, gs1)
    g0.wait()
    w0 = pltpu.async_copy(buf0, g1_hbm.at[pl.ds(wid * TPW, TPW)], ws0)
    g1c.wait()
    w1 = pltpu.async_copy(buf1, g2_hbm.at[pl.ds(wid * TPW, TPW)], ws1)
    w0.wait()
    w1.wait()


_HMASK = np.int32(-65536)


def _unpack(xp):
    lo = lax.bitcast_convert_type(lax.shift_left(xp, 16), jnp.float32)
    hi = lax.bitcast_convert_type(xp & _HMASK, jnp.float32)
    return lo, hi


def _pack(lo, hi):
    loi = lax.bitcast_convert_type(_bf(lo), jnp.int32)
    hii = lax.bitcast_convert_type(_bf(hi), jnp.int32)
    return lax.shift_right_logical(loi, 16) | hii


def _expert_body(be_ref, xg_ref, w1_ref, w2_ref, h_ref):
    lo, hi = _unpack(xg_ref[...])
    xb = jnp.concatenate([lo, hi], axis=1).astype(jnp.bfloat16)
    h = jnp.dot(xb, w1_ref[0], preferred_element_type=jnp.float32)
    h = h / (1.0 + jnp.exp(-h))
    o = jnp.dot(h.astype(jnp.bfloat16), w2_ref[0],
                preferred_element_type=jnp.float32)
    h_ref[...] = _pack(o[:, :D // 2], o[:, D // 2:])


def _final_body(y_ref, g1_ref, g2_ref, gates_ref, postmoe_ref, out_ref):
    g = gates_ref[...]
    a_lo, a_hi = _unpack(g1_ref[...])
    b_lo, b_hi = _unpack(g2_ref[...])
    moe = jnp.concatenate(
        [g[:, 0:1] * a_lo + g[:, 1:2] * b_lo,
         g[:, 0:1] * a_hi + g[:, 1:2] * b_hi], axis=1)
    out_ref[...] = y_ref[...] + _rms(moe, postmoe_ref[...])


def kernel(x, pre_mqa_scale, post_mqa_scale, pre_moe_scale, post_moe_scale,
           Wq, Wk, Wv, Wo, Wr, w1, w2):
    xs = x.reshape(S, D).astype(jnp.float32)
    pres = pre_mqa_scale.reshape(1, D)
    posts = post_mqa_scale.reshape(1, D)
    premoe = pre_moe_scale.reshape(1, D)
    postmoe = post_moe_scale.reshape(1, D)
    w1b = w1.astype(jnp.bfloat16)
    w2b = w2.astype(jnp.bfloat16)

    full = lambda shape: pl.BlockSpec(shape, lambda i: (0,) * len(shape))
    y, xnb, w_all, mask = pl.pallas_call(
        _attn_body,
        grid=(S // SB,),
        in_specs=[
            pl.BlockSpec((SB, D), lambda i: (i, 0)),
            full((1, D)), full((1, D)), full((1, D)),
            full((D, H * DH)), full((D, KV * DH)), full((D, KV * DH)),
            full((H * DH, D)), full((D, E)),
        ],
        out_specs=[
            pl.BlockSpec((SB, D), lambda i: (i, 0)),
            pl.BlockSpec((SB, D // 2), lambda i: (i, 0)),
            pl.BlockSpec((SB, E), lambda i: (i, 0)),
            pl.BlockSpec((SB, E), lambda i: (i, 0)),
        ],
        out_shape=[
            jax.ShapeDtypeStruct((S, D), jnp.float32),
            jax.ShapeDtypeStruct((S, D // 2), jnp.int32),
            jax.ShapeDtypeStruct((S, E), jnp.float32),
            jax.ShapeDtypeStruct((S, E), jnp.float32),
        ],
    )(xs, pres, posts, premoe, Wq, Wk, Wv, Wo, Wr)

    pos2, gates2, beM = pl.pallas_call(
        _route_body,
        grid=(1,),
        in_specs=[full((S, E))],
        out_specs=[full((S, 2)), full((S, 2)), full((NB, E))],
        out_shape=[
            jax.ShapeDtypeStruct((S, 2), jnp.int32),
            jax.ShapeDtypeStruct((S, 2), jnp.float32),
            jax.ShapeDtypeStruct((NB, E), jnp.float32),
        ],
    )(w_all)

    blk_expert = beM[:, 0].astype(jnp.int32)
    p1_2d = pos2[:, 0].reshape(NW, TPW)
    p2_2d = pos2[:, 1].reshape(NW, TPW)

    xg = _sc_scatter_rows(p1_2d, p2_2d, xnb)

    grid_spec = pltpu.PrefetchScalarGridSpec(
        num_scalar_prefetch=1,
        grid=(NB,),
        in_specs=[
            pl.BlockSpec((BLK, D // 2), lambda i, be: (i, 0)),
            pl.BlockSpec((1, D, DFF), lambda i, be: (be[i], 0, 0)),
            pl.BlockSpec((1, DFF, D), lambda i, be: (be[i], 0, 0)),
        ],
        out_specs=pl.BlockSpec((BLK, D // 2), lambda i, be: (i, 0)),
    )
    hrows = pl.pallas_call(
        _expert_body,
        grid_spec=grid_spec,
        out_shape=jax.ShapeDtypeStruct((NPAD, D // 2), jnp.int32),
    )(blk_expert, xg, w1b, w2b)

    g1, g2 = _sc_gather_back(p1_2d, p2_2d, hrows)

    out = pl.pallas_call(
        _final_body,
        grid=(S // SB,),
        in_specs=[
            pl.BlockSpec((SB, D), lambda i: (i, 0)),
            pl.BlockSpec((SB, D // 2), lambda i: (i, 0)),
            pl.BlockSpec((SB, D // 2), lambda i: (i, 0)),
            pl.BlockSpec((SB, 2), lambda i: (i, 0)),
            full((1, D)),
        ],
        out_specs=pl.BlockSpec((SB, D), lambda i: (i, 0)),
        out_shape=jax.ShapeDtypeStruct((S, D), jnp.float32),
    )(y, g1, g2, gates2, postmoe)

    return out.reshape(1, S, D), mask


# speedup vs baseline: 3.2793x; 1.0072x over previous
import functools
import jax
import jax.numpy as jnp
import numpy as np
from jax import lax
from jax.experimental import pallas as pl
from jax.experimental.pallas import tpu as pltpu
from jax.experimental.pallas import tpu_sc as plsc

S = 2048
D = 1024
H = 16
KV = 2
DH = 64
E = 8
DFF = 2048
EPS = 1e-05

SB = 256
BLK = 256
NB = 24
NPAD = NB * BLK
NA = 2 * S
NW = 32
RPW = NPAD // NW


def _rms(v, scale, eps=EPS):
    ms = jnp.mean(v * v, axis=-1, keepdims=True)
    return v * lax.rsqrt(ms + eps) * scale


def _f32(v):
    return v.astype(jnp.float32)


def _bf(v):
    return _f32(v.astype(jnp.bfloat16))


def _segsum(p, Sseg):
    hi = _bf(p)
    r1 = p - hi
    hi2 = _bf(r1)
    hi3 = _bf(r1 - hi2)
    acc = jnp.dot(hi, Sseg, preferred_element_type=jnp.float32)
    acc += jnp.dot(hi2, Sseg, preferred_element_type=jnp.float32)
    acc += jnp.dot(hi3, Sseg, preferred_element_type=jnp.float32)
    return acc


def _attn_body(xb_ref, pres_ref, posts_ref, premoe_ref, Wq_ref, Wk_ref,
               Wv_ref, Wo_ref, Wr_ref, y_ref, xn2b_ref, mask_ref,
               pos_ref, gates_ref, be_ref, w_acc):
    blk = pl.program_id(0)
    xb = xb_ref[...]
    xn = _rms(xb, pres_ref[...])
    q = jnp.dot(xn, Wq_ref[...], preferred_element_type=jnp.float32)
    kk = jnp.dot(xn, Wk_ref[...], preferred_element_type=jnp.float32)
    vv = jnp.dot(xn, Wv_ref[...], preferred_element_type=jnp.float32)
    k0, k1 = kk[:, :DH], kk[:, DH:]
    v0, v1 = vv[:, :DH], vv[:, DH:]
    krep0 = jnp.concatenate([k0] * H, axis=1)
    krep1 = jnp.concatenate([k1] * H, axis=1)
    r_i = lax.broadcasted_iota(jnp.int32, (H * DH, H), 0)
    c_i = lax.broadcasted_iota(jnp.int32, (H * DH, H), 1)
    Sseg = ((r_i // DH) == c_i).astype(jnp.float32)
    scale = 1.0 / np.sqrt(DH)
    qb = _bf(q)
    A0 = _segsum(qb * _bf(krep0), Sseg) * scale
    A1 = _segsum(qb * _bf(krep1), Sseg) * scale
    m = jnp.maximum(A0, A1)
    h_i = lax.broadcasted_iota(jnp.int32, (SB, H), 1)
    n0 = jnp.minimum(h_i + 1, 8).astype(jnp.float32)
    n1 = jnp.maximum(h_i - 7, 0).astype(jnp.float32)
    e0 = jnp.exp(A0 - m)
    e1 = jnp.exp(A1 - m)
    z = n0 * e0 + n1 * e1
    w0 = e0 / z
    w1 = e1 / z
    rr = lax.broadcasted_iota(jnp.int32, (H, H * DH), 0)
    cc = lax.broadcasted_iota(jnp.int32, (H, H * DH), 1)
    Eseg = ((cc // DH) == rr).astype(jnp.float32)
    W0 = jnp.dot(_bf(w0), Eseg, preferred_element_type=jnp.float32)
    W1 = jnp.dot(_bf(w1), Eseg, preferred_element_type=jnp.float32)
    vb0 = _bf(jnp.concatenate([v0] * H, axis=1))
    vb1 = _bf(jnp.concatenate([v1] * H, axis=1))
    l_i = lax.broadcasted_iota(jnp.int32, (SB, H * DH), 1) // DH
    n0f = jnp.minimum(l_i + 1, 8).astype(jnp.float32)
    n1f = jnp.maximum(l_i - 7, 0).astype(jnp.float32)
    attn = n0f * (W0 * vb0) + n1f * (W1 * vb1)
    mqa = jnp.dot(attn, Wo_ref[...], preferred_element_type=jnp.float32)
    y = xb + _rms(mqa, posts_ref[...])
    y_ref[...] = y
    xn2 = _rms(y, premoe_ref[...])
    lo = lax.bitcast_convert_type(_bf(xn2[:, :D // 2]), jnp.int32)
    hi = lax.bitcast_convert_type(_bf(xn2[:, D // 2:]), jnp.int32)
    xn2b_ref[...] = lax.shift_right_logical(lo, 16) | hi
    logits = jnp.dot(xn2, Wr_ref[...], preferred_element_type=jnp.float32)
    lm = jnp.max(logits, axis=1, keepdims=True)
    p = jnp.exp(logits - lm)
    p = p / jnp.sum(p, axis=1, keepdims=True)
    e_i = lax.broadcasted_iota(jnp.int32, (SB, E), 1)
    m1 = jnp.max(p, axis=1, keepdims=True)
    i1 = jnp.min(jnp.where(p == m1, e_i, E), axis=1, keepdims=True)
    p2 = jnp.where(e_i == i1, -1.0, p)
    m2 = jnp.max(p2, axis=1, keepdims=True)
    i2 = jnp.min(jnp.where(p2 == m2, e_i, E), axis=1, keepdims=True)
    tv1 = m1 / (m1 + m2)
    tv2 = m2 / (m1 + m2)
    w_all = (jnp.where(e_i == i1, tv1, 0.0) + jnp.where(e_i == i2, tv2, 0.0))
    w_acc[pl.ds(blk * SB, SB), :] = w_all
    mask_ref[...] = (w_all > 0).astype(jnp.float32)

    @pl.when(blk == S // SB - 1)
    def _route():
        _route_tail(w_acc[...], pos_ref, gates_ref, be_ref)


def _route_tail(w_all, pos_ref, gates_ref, be_ref):
    maskb = w_all > 0.0
    maskf = maskb.astype(jnp.float32)
    r_i = lax.broadcasted_iota(jnp.int32, (S, S), 0)
    c_i = lax.broadcasted_iota(jnp.int32, (S, S), 1)
    L = (c_i < r_i).astype(jnp.float32)
    rank = jnp.dot(L, maskf, preferred_element_type=jnp.float32)
    counts = jnp.sum(maskf, axis=0, keepdims=True)
    nb = jnp.floor((counts + (BLK - 1)) * (1.0 / BLK))
    rr = lax.broadcasted_iota(jnp.int32, (E, E), 0)
    cc = lax.broadcasted_iota(jnp.int32, (E, E), 1)
    T8 = (rr < cc).astype(jnp.float32)
    starts = jnp.dot(nb, T8, preferred_element_type=jnp.float32)
    off = starts * float(BLK)
    posf = off + rank
    e_i = lax.broadcasted_iota(jnp.int32, (S, E), 1)
    e1 = jnp.min(jnp.where(maskb, e_i, E), axis=1, keepdims=True)
    e2 = jnp.max(jnp.where(maskb, e_i, -1), axis=1, keepdims=True)
    sel1 = e_i == e1
    sel2 = e_i == e2
    p1 = jnp.sum(jnp.where(sel1, posf, 0.0), axis=1, keepdims=True)
    p2 = jnp.sum(jnp.where(sel2, posf, 0.0), axis=1, keepdims=True)
    g1 = jnp.sum(jnp.where(sel1, w_all, 0.0), axis=1, keepdims=True)
    g2 = jnp.sum(jnp.where(sel2, w_all, 0.0), axis=1, keepdims=True)
    pos_ref[...] = jnp.concatenate([p1, p2], axis=1).astype(jnp.int32)
    gates_ref[...] = jnp.concatenate([g1, g2], axis=1)
    b_i = lax.broadcasted_iota(jnp.int32, (NB, E), 0).astype(jnp.float32)
    ge = (b_i >= starts).astype(jnp.float32)
    be = jnp.sum(ge, axis=1, keepdims=True) - 1.0
    be_ref[...] = jnp.broadcast_to(be, (NB, E))


_SC_MESH = plsc.VectorSubcoreMesh(core_axis_name="c", subcore_axis_name="s")
DP = D // 2
TPW = S // NW


@functools.partial(
    pl.kernel, mesh=_SC_MESH,
    out_type=jax.ShapeDtypeStruct((NPAD, DP), jnp.int32),
    scratch_types=[
        pltpu.VMEM((1, TPW), jnp.int32),
        pltpu.VMEM((1, TPW), jnp.int32),
        pltpu.VMEM((TPW, DP), jnp.int32),
        pltpu.SemaphoreType.DMA,
        pltpu.SemaphoreType.DMA,
        pltpu.SemaphoreType.DMA,
        pltpu.SemaphoreType.DMA,
        pltpu.SemaphoreType.DMA,
    ],
)
def _sc_scatter_rows(p1_hbm, p2_hbm, xp_hbm, xg_hbm, idx1, idx2, buf,
                     ls, is1, is2, ws1, ws2):
    c = lax.axis_index("c")
    s = lax.axis_index("s")
    wid = s * 2 + c
    l0 = pltpu.async_copy(xp_hbm.at[pl.ds(wid * TPW, TPW)], buf, ls)
    l1 = pltpu.async_copy(p1_hbm.at[pl.ds(wid, 1)], idx1, is1)
    l2 = pltpu.async_copy(p2_hbm.at[pl.ds(wid, 1)], idx2, is2)
    l0.wait()
    l1.wait()
    l2.wait()
    w1 = pltpu.async_copy(buf, xg_hbm.at[idx1.at[0]], ws1)
    w2 = pltpu.async_copy(buf, xg_hbm.at[idx2.at[0]], ws2)
    w1.wait()
    w2.wait()


@functools.partial(
    pl.kernel, mesh=_SC_MESH,
    out_type=[
        jax.ShapeDtypeStruct((S, DP), jnp.int32),
        jax.ShapeDtypeStruct((S, DP), jnp.int32),
    ],
    scratch_types=[
        pltpu.VMEM((1, TPW), jnp.int32),
        pltpu.VMEM((1, TPW), jnp.int32),
        pltpu.VMEM((TPW, DP), jnp.int32),
        pltpu.VMEM((TPW, DP), jnp.int32),
        pltpu.SemaphoreType.DMA,
        pltpu.SemaphoreType.DMA,
        pltpu.SemaphoreType.DMA,
        pltpu.SemaphoreType.DMA,
        pltpu.SemaphoreType.DMA,
        pltpu.SemaphoreType.DMA,
    ],
)
def _sc_gather_back(p1_hbm, p2_hbm, h_hbm, g1_hbm, g2_hbm, idx1, idx2,
                    buf0, buf1, is1, is2, gs0, gs1, ws0, ws1):
    c = lax.axis_index("c")
    s = lax.axis_index("s")
    wid = s * 2 + c
    l1 = pltpu.async_copy(p1_hbm.at[pl.ds(wid, 1)], idx1, is1)
    l2 = pltpu.async_copy(p2_hbm.at[pl.ds(wid, 1)], idx2, is2)
    l1.wait()
    l2.wait()
    g0 = pltpu.async_copy(h_hbm.at[idx1.at[0]], buf0, gs0)
    g1c = pltpu.async_copy(h_hbm.at[idx2.at[0]], buf1, gs1)
    g0.wait()
    w0 = pltpu.async_copy(buf0, g1_hbm.at[pl.ds(wid * TPW, TPW)], ws0)
    g1c.wait()
    w1 = pltpu.async_copy(buf1, g2_hbm.at[pl.ds(wid * TPW, TPW)], ws1)
    w0.wait()
    w1.wait()


_HMASK = np.int32(-65536)


def _unpack(xp):
    lo = lax.bitcast_convert_type(lax.shift_left(xp, 16), jnp.float32)
    hi = lax.bitcast_convert_type(xp & _HMASK, jnp.float32)
    return lo, hi


def _pack(lo, hi):
    loi = lax.bitcast_convert_type(_bf(lo), jnp.int32)
    hii = lax.bitcast_convert_type(_bf(hi), jnp.int32)
    return lax.shift_right_logical(loi, 16) | hii


def _expert_body(be_ref, xg_ref, w1_ref, w2_ref, h_ref):
    lo, hi = _unpack(xg_ref[...])
    xb = jnp.concatenate([lo, hi], axis=1).astype(jnp.bfloat16)
    h = jnp.dot(xb, w1_ref[0], preferred_element_type=jnp.float32)
    h = h / (1.0 + jnp.exp(-h))
    o = jnp.dot(h.astype(jnp.bfloat16), w2_ref[0],
                preferred_element_type=jnp.float32)
    h_ref[...] = _pack(o[:, :D // 2], o[:, D // 2:])


def _final_body(y_ref, g1_ref, g2_ref, gates_ref, postmoe_ref, out_ref):
    g = gates_ref[...]
    a_lo, a_hi = _unpack(g1_ref[...])
    b_lo, b_hi = _unpack(g2_ref[...])
    moe = jnp.concatenate(
        [g[:, 0:1] * a_lo + g[:, 1:2] * b_lo,
         g[:, 0:1] * a_hi + g[:, 1:2] * b_hi], axis=1)
    out_ref[...] = y_ref[...] + _rms(moe, postmoe_ref[...])


def kernel(x, pre_mqa_scale, post_mqa_scale, pre_moe_scale, post_moe_scale,
           Wq, Wk, Wv, Wo, Wr, w1, w2):
    xs = x.reshape(S, D).astype(jnp.float32)
    pres = pre_mqa_scale.reshape(1, D)
    posts = post_mqa_scale.reshape(1, D)
    premoe = pre_moe_scale.reshape(1, D)
    postmoe = post_moe_scale.reshape(1, D)
    w1b = w1.astype(jnp.bfloat16)
    w2b = w2.astype(jnp.bfloat16)

    full = lambda shape: pl.BlockSpec(shape, lambda i: (0,) * len(shape))
    y, xnb, mask, pos2, gates2, beM = pl.pallas_call(
        _attn_body,
        grid=(S // SB,),
        in_specs=[
            pl.BlockSpec((SB, D), lambda i: (i, 0)),
            full((1, D)), full((1, D)), full((1, D)),
            full((D, H * DH)), full((D, KV * DH)), full((D, KV * DH)),
            full((H * DH, D)), full((D, E)),
        ],
        out_specs=[
            pl.BlockSpec((SB, D), lambda i: (i, 0)),
            pl.BlockSpec((SB, D // 2), lambda i: (i, 0)),
            pl.BlockSpec((SB, E), lambda i: (i, 0)),
            full((S, 2)), full((S, 2)), full((NB, E)),
        ],
        out_shape=[
            jax.ShapeDtypeStruct((S, D), jnp.float32),
            jax.ShapeDtypeStruct((S, D // 2), jnp.int32),
            jax.ShapeDtypeStruct((S, E), jnp.float32),
            jax.ShapeDtypeStruct((S, 2), jnp.int32),
            jax.ShapeDtypeStruct((S, 2), jnp.float32),
            jax.ShapeDtypeStruct((NB, E), jnp.float32),
        ],
        scratch_shapes=[pltpu.VMEM((S, E), jnp.float32)],
    )(xs, pres, posts, premoe, Wq, Wk, Wv, Wo, Wr)

    blk_expert = beM[:, 0].astype(jnp.int32)
    p1_2d = pos2[:, 0].reshape(NW, TPW)
    p2_2d = pos2[:, 1].reshape(NW, TPW)

    xg = _sc_scatter_rows(p1_2d, p2_2d, xnb)

    grid_spec = pltpu.PrefetchScalarGridSpec(
        num_scalar_prefetch=1,
        grid=(NB,),
        in_specs=[
            pl.BlockSpec((BLK, D // 2), lambda i, be: (i, 0)),
            pl.BlockSpec((1, D, DFF), lambda i, be: (be[i], 0, 0)),
            pl.BlockSpec((1, DFF, D), lambda i, be: (be[i], 0, 0)),
        ],
        out_specs=pl.BlockSpec((BLK, D // 2), lambda i, be: (i, 0)),
    )
    hrows = pl.pallas_call(
        _expert_body,
        grid_spec=grid_spec,
        out_shape=jax.ShapeDtypeStruct((NPAD, D // 2), jnp.int32),
    )(blk_expert, xg, w1b, w2b)

    g1, g2 = _sc_gather_back(p1_2d, p2_2d, hrows)

    out = pl.pallas_call(
        _final_body,
        grid=(S // SB,),
        in_specs=[
            pl.BlockSpec((SB, D), lambda i: (i, 0)),
            pl.BlockSpec((SB, D // 2), lambda i: (i, 0)),
            pl.BlockSpec((SB, D // 2), lambda i: (i, 0)),
            pl.BlockSpec((SB, 2), lambda i: (i, 0)),
            full((1, D)),
        ],
        out_specs=pl.BlockSpec((SB, D), lambda i: (i, 0)),
        out_shape=jax.ShapeDtypeStruct((S, D), jnp.float32),
    )(y, g1, g2, gates2, postmoe)

    return out.reshape(1, S, D), mask
